# baseline (speedup 1.0000x reference)
import functools

import jax
import jax.numpy as jnp
from jax import lax
from jax.experimental import pallas as pl
from jax.experimental.pallas import tpu as pltpu
from jax.experimental.pallas import tpu_sc as plsc

N = 10000
E = 320000
H = 128
D_EDGE = 16
L = 3

NC = 2
NS = 16
NW = NC * NS
GROUP = 64
GPW = 160
EW = GROUP * GPW
EPAD = EW * NW
NPAD = 10112
RPW = NPAD // NS
KG = 8

BN = 2000
BE = 4096


def _sc_body(h_hbm, emb_hbm, src_hbm, dst_hbm, zero_hbm, out_hbm,
             srcbuf, dstbuf, gbuf0, gbuf1, mbuf0, mbuf1, sbuf0, sbuf1, acc,
             semg0, semg1, semm0, semm1, sems0, sems1):
    c = lax.axis_index("c")
    s = lax.axis_index("s")
    wid = c * NS + s
    gbufs = (gbuf0, gbuf1)
    mbufs = (mbuf0, mbuf1)
    sbufs = (sbuf0, sbuf1)
    semgs = (semg0, semg1)
    semms = (semm0, semm1)
    semss = (sems0, sems1)

    pltpu.sync_copy(zero_hbm, acc.at[pl.ds(s * RPW, RPW)])

    plsc.subcore_barrier()

    def start(cc, j):
        ebase2 = pl.multiple_of(
            wid * (EW // 2) + (cc * KG + j) * (GROUP // 2), GROUP // 2)
        dm = pltpu.async_copy(emb_hbm.at[pl.ds(ebase2, GROUP // 2)],
                              mbufs[j % 2], semms[j % 2])
        dg = pltpu.async_copy(h_hbm.at[srcbuf.at[j]], gbufs[j % 2],
                              semgs[j % 2])
        return dm, dg

    def compute(j):
        gbuf = gbufs[j % 2]
        mbuf = mbufs[j % 2]
        sbuf = sbufs[j % 2]
        mask = jnp.int32(-65536)
        two16 = jnp.int32(65536)

        def _row(r, cr):
            for half in range(2):
                e = 2 * r + half
                for k in range(H // 32):
                    mw = mbuf[r, pl.ds(64 * half + 16 * k, 16)]
                    mlo = lax.bitcast_convert_type(mw * two16, jnp.float32)
                    mhi = lax.bitcast_convert_type(mw & mask, jnp.float32)
                    sl0 = pl.ds(16 * k, 16)
                    sl1 = pl.ds(64 + 16 * k, 16)
                    sbuf[e, sl0] = jnp.maximum(gbuf[e, sl0] + mlo, 0.0)
                    sbuf[e, sl1] = jnp.maximum(gbuf[e, sl1] + mhi, 0.0)
            return cr

        lax.fori_loop(0, GROUP // 2, _row, 0)

    def chunk_step(cc, carry0):
        pltpu.sync_copy(src_hbm.at[pl.ds(wid * GPW + cc * KG, KG)], srcbuf)
        pltpu.sync_copy(dst_hbm.at[pl.ds(wid * GPW + cc * KG, KG)], dstbuf)

        lds = [None] * KG
        scs = [None] * KG
        lds[0] = start(cc, 0)
        lds[1] = start(cc, 1)
        for k in range(KG):
            dm, dg = lds[k]
            dm.wait()
            dg.wait()
            if k >= 2:
                scs[k - 2].wait()
            compute(k)
            scs[k] = pltpu.async_copy(sbufs[k % 2], acc.at[dstbuf.at[k]],
                                      semss[k % 2], add=True)
            if k + 2 < KG:
                lds[k + 2] = start(cc, k + 2)
        scs[KG - 2].wait()
        scs[KG - 1].wait()
        return carry0

    lax.fori_loop(0, GPW // KG, chunk_step, 0)
    plsc.subcore_barrier()
    pltpu.sync_copy(acc.at[pl.ds(s * RPW, RPW)],
                    out_hbm.at[c, pl.ds(s * RPW, RPW)])


_sc_gather_scatter = functools.partial(
    pl.kernel,
    out_type=jax.ShapeDtypeStruct((NC, NPAD, H), jnp.float32),
    mesh=plsc.VectorSubcoreMesh(
        core_axis_name="c", subcore_axis_name="s",
        num_cores=NC, num_subcores=NS),
    scratch_types=[
        pltpu.VMEM((KG, GROUP), jnp.int32),
        pltpu.VMEM((KG, GROUP), jnp.int32),
        pltpu.VMEM((GROUP, H), jnp.float32),
        pltpu.VMEM((GROUP, H), jnp.float32),
        pltpu.VMEM((GROUP // 2, H), jnp.int32),
        pltpu.VMEM((GROUP // 2, H), jnp.int32),
        pltpu.VMEM((GROUP, H), jnp.float32),
        pltpu.VMEM((GROUP, H), jnp.float32),
        pltpu.VMEM_SHARED((NPAD, H), jnp.float32),
        pltpu.SemaphoreType.DMA,
        pltpu.SemaphoreType.DMA,
        pltpu.SemaphoreType.DMA,
        pltpu.SemaphoreType.DMA,
        pltpu.SemaphoreType.DMA,
        pltpu.SemaphoreType.DMA,
    ],
)(_sc_body)


def _enc_body(x_ref, w_ref, b_ref, o_ref):
    o_ref[...] = (
        jnp.dot(x_ref[...], w_ref[...], preferred_element_type=jnp.float32)
        + b_ref[...])


_encoder = pl.pallas_call(
    _enc_body,
    grid=(N // BN,),
    in_specs=[
        pl.BlockSpec((BN, 128), lambda i: (i, 0)),
        pl.BlockSpec((128, H), lambda i: (0, 0)),
        pl.BlockSpec((1, H), lambda i: (0, 0)),
    ],
    out_specs=pl.BlockSpec((BN, H), lambda i: (i, 0)),
    out_shape=jax.ShapeDtypeStruct((N, H), jnp.float32),
)


def _pack_bf16(z):
    zb = z.astype(jnp.bfloat16).astype(jnp.float32)
    zi = lax.bitcast_convert_type(zb, jnp.int32)
    return lax.shift_right_logical(zi[:, :64], 16) | zi[:, 64:]


def _edge_body(a_ref, w_ref, b_ref, o_ref):
    za = (jnp.dot(a_ref[:, :D_EDGE], w_ref[...],
                  preferred_element_type=jnp.float32) + b_ref[...])
    zb = (jnp.dot(a_ref[:, D_EDGE:], w_ref[...],
                  preferred_element_type=jnp.float32) + b_ref[...])
    o_ref[...] = jnp.concatenate([_pack_bf16(za), _pack_bf16(zb)], axis=1)


BE2 = 2048

_edge_embed = pl.pallas_call(
    _edge_body,
    grid=(EPAD // 2 // BE2,),
    in_specs=[
        pl.BlockSpec((BE2, 2 * D_EDGE), lambda i: (i, 0)),
        pl.BlockSpec((D_EDGE, H), lambda i: (0, 0)),
        pl.BlockSpec((1, H), lambda i: (0, 0)),
    ],
    out_specs=pl.BlockSpec((BE2, H), lambda i: (i, 0)),
    out_shape=jax.ShapeDtypeStruct((EPAD // 2, H), jnp.int32),
)


def _node_body(eps_ref, h_ref, a_ref, w1_ref, b1_ref, w2_ref, b2_ref,
               o_ref):
    z = h_ref[...] * eps_ref[0, 0] + a_ref[0] + a_ref[1]
    z = jnp.maximum(
        jnp.dot(z, w1_ref[...], preferred_element_type=jnp.float32)
        + b1_ref[...], 0.0)
    z = (jnp.dot(z, w2_ref[...], preferred_element_type=jnp.float32)
         + b2_ref[...])
    o_ref[...] = jnp.maximum(z, 0.0)


_node_update = pl.pallas_call(
    _node_body,
    grid=(N // BN,),
    in_specs=[
        pl.BlockSpec(memory_space=pltpu.SMEM),
        pl.BlockSpec((BN, H), lambda i: (i, 0)),
        pl.BlockSpec((NC, BN, H), lambda i: (0, i, 0)),
        pl.BlockSpec((H, H), lambda i: (0, 0)),
        pl.BlockSpec((1, H), lambda i: (0, 0)),
        pl.BlockSpec((H, H), lambda i: (0, 0)),
        pl.BlockSpec((1, H), lambda i: (0, 0)),
    ],
    out_specs=pl.BlockSpec((BN, H), lambda i: (i, 0)),
    out_shape=jax.ShapeDtypeStruct((N, H), jnp.float32),
)


def _head_body(h_ref, w0_ref, b0_ref, w1_ref, b1_ref, w2_ref, b2_ref, o_ref):
    o = jnp.maximum(
        jnp.dot(h_ref[...], w0_ref[...], preferred_element_type=jnp.float32)
        + b0_ref[...], 0.0)
    o = jnp.maximum(
        jnp.dot(o, w1_ref[...], preferred_element_type=jnp.float32)
        + b1_ref[...], 0.0)
    o_ref[...] = (
        jnp.dot(o, w2_ref[...], preferred_element_type=jnp.float32)
        + b2_ref[...])


_head = pl.pallas_call(
    _head_body,
    grid=(N // BN,),
    in_specs=[
        pl.BlockSpec((BN, H), lambda i: (i, 0)),
        pl.BlockSpec((H, H), lambda i: (0, 0)),
        pl.BlockSpec((1, H), lambda i: (0, 0)),
        pl.BlockSpec((H, H), lambda i: (0, 0)),
        pl.BlockSpec((1, H), lambda i: (0, 0)),
        pl.BlockSpec((H, H), lambda i: (0, 0)),
        pl.BlockSpec((1, H), lambda i: (0, 0)),
    ],
    out_specs=pl.BlockSpec((BN, H), lambda i: (i, 0)),
    out_shape=jax.ShapeDtypeStruct((N, H), jnp.float32),
)


def kernel(x, edge_index, edge_attr, y, params):
    p = params
    pad = EPAD - E
    src2d = jnp.concatenate(
        [edge_index[0], jnp.zeros((pad,), jnp.int32)]).reshape(EPAD // GROUP, GROUP)
    dst2d = jnp.concatenate(
        [edge_index[1], jnp.full((pad,), N, jnp.int32)]).reshape(EPAD // GROUP, GROUP)
    ea2 = jnp.concatenate(
        [edge_attr, jnp.zeros((pad, D_EDGE), jnp.float32)],
        axis=0).reshape(EPAD // 2, 2 * D_EDGE)
    zero_rows = jnp.zeros((RPW, H), jnp.float32)

    h = _encoder(x, p['enc_Wn'], p['enc_bn'].reshape(1, H))
    embs = []
    for l in range(L):
        wc = p['enc_We'] @ p[f'l{l}_elin_W']
        bc = p['enc_be'] @ p[f'l{l}_elin_W'] + p[f'l{l}_elin_b']
        embs.append(_edge_embed(ea2, wc, bc.reshape(1, H)))
    for l in range(L):
        agg2 = _sc_gather_scatter(h, embs[l], src2d, dst2d, zero_rows)
        g = p[f'l{l}_bn_g']
        w2 = p[f'l{l}_W2'] * g[None, :]
        b2 = p[f'l{l}_b2'] * g + p[f'l{l}_bn_b']
        epsm = (1.0 + p[f'l{l}_eps']).reshape(1, 1)
        h = _node_update(epsm, h, agg2, p[f'l{l}_W1'],
                         p[f'l{l}_b1'].reshape(1, H), w2, b2.reshape(1, H))

    w2p = jnp.pad(p['head_W2'], ((0, 0), (0, 127)))
    b2p = jnp.pad(p['head_b2'], (0, 127)).reshape(1, 128)
    o = _head(h, p['head_W0'], p['head_b0'].reshape(1, H),
              p['head_W1'], p['head_b1'].reshape(1, H), w2p, b2p)
    pred = o[:, :1]

    true_class = jnp.full((N,), -1, jnp.int32)
    true_label = jnp.where(y != -1.0, y, -1.0)
    return (pred, true_class, true_label)


# speedup vs baseline: 1.0895x; 1.0895x over previous
import functools

import jax
import jax.numpy as jnp
from jax import lax
from jax.experimental import pallas as pl
from jax.experimental.pallas import tpu as pltpu
from jax.experimental.pallas import tpu_sc as plsc

N = 10000
E = 320000
H = 128
D_EDGE = 16
L = 3

NC = 2
NS = 16
NW = NC * NS
GROUP = 64
GPW = 160
EW = GROUP * GPW
EPAD = EW * NW
NPAD = 10112
RPW = NPAD // NS
KG = 8

BN = 2000
BE = 4096


def _sc_body(h_hbm, emb_hbm, src_hbm, dst_hbm, zero_hbm, out_hbm,
             srcbuf, dstbuf,
             gbuf0, gbuf1, mbuf0, mbuf1, mbuf2, acc,
             semg0, semg1, semm0, semm1, semm2, sems0, sems1, sems2):
    c = lax.axis_index("c")
    s = lax.axis_index("s")
    wid = c * NS + s
    gbufs = (gbuf0, gbuf1)
    mbufs = (mbuf0, mbuf1, mbuf2)
    semgs = (semg0, semg1)
    semms = (semm0, semm1, semm2)
    semss = (sems0, sems1, sems2)

    pltpu.sync_copy(zero_hbm, acc.at[pl.ds(s * RPW, RPW)])

    plsc.subcore_barrier()

    def start(cc, j):
        ebase = wid * EW + (cc * KG + j) * GROUP
        dm = pltpu.async_copy(emb_hbm.at[pl.ds(ebase, GROUP)],
                              mbufs[j % 3], semms[j % 3])
        dg = pltpu.async_copy(h_hbm.at[srcbuf.at[j]], gbufs[j % 2],
                              semgs[j % 2])
        return dm, dg

    def compute(j):
        gbuf = gbufs[j % 2]
        mbuf = mbufs[j % 3]

        def _row(i, cr):
            for k in range(H // 16):
                sl = pl.ds(k * 16, 16)
                mbuf[i, sl] = jnp.maximum(mbuf[i, sl] + gbuf[i, sl], 0.0)
            return cr

        lax.fori_loop(0, GROUP, _row, 0)

    def chunk_step(cc, carry0):
        pltpu.sync_copy(src_hbm.at[pl.ds(wid * GPW + cc * KG, KG)], srcbuf)
        pltpu.sync_copy(dst_hbm.at[pl.ds(wid * GPW + cc * KG, KG)], dstbuf)

        lds = [None] * KG
        scs = [None] * KG
        lds[0] = start(cc, 0)
        lds[1] = start(cc, 1)
        for k in range(KG):
            dm, dg = lds[k]
            dm.wait()
            dg.wait()
            compute(k)
            scs[k] = pltpu.async_copy(mbufs[k % 3], acc.at[dstbuf.at[k]],
                                      semss[k % 3], add=True)
            if k + 2 < KG:
                if k >= 1:
                    scs[k - 1].wait()
                lds[k + 2] = start(cc, k + 2)
        for k in range(KG - 3, KG):
            scs[k].wait()
        return carry0

    lax.fori_loop(0, GPW // KG, chunk_step, 0)
    plsc.subcore_barrier()
    pltpu.sync_copy(acc.at[pl.ds(s * RPW, RPW)],
                    out_hbm.at[c, pl.ds(s * RPW, RPW)])


_sc_gather_scatter = functools.partial(
    pl.kernel,
    out_type=jax.ShapeDtypeStruct((NC, NPAD, H), jnp.float32),
    mesh=plsc.VectorSubcoreMesh(
        core_axis_name="c", subcore_axis_name="s",
        num_cores=NC, num_subcores=NS),
    scratch_types=[
        pltpu.VMEM((KG, GROUP), jnp.int32),
        pltpu.VMEM((KG, GROUP), jnp.int32),
        pltpu.VMEM((GROUP, H), jnp.float32),
        pltpu.VMEM((GROUP, H), jnp.float32),
        pltpu.VMEM((GROUP, H), jnp.float32),
        pltpu.VMEM((GROUP, H), jnp.float32),
        pltpu.VMEM((GROUP, H), jnp.float32),
        pltpu.VMEM_SHARED((NPAD, H), jnp.float32),
        pltpu.SemaphoreType.DMA,
        pltpu.SemaphoreType.DMA,
        pltpu.SemaphoreType.DMA,
        pltpu.SemaphoreType.DMA,
        pltpu.SemaphoreType.DMA,
        pltpu.SemaphoreType.DMA,
        pltpu.SemaphoreType.DMA,
        pltpu.SemaphoreType.DMA,
    ],
)(_sc_body)


def _enc_body(x_ref, w_ref, b_ref, o_ref):
    o_ref[...] = (
        jnp.dot(x_ref[...], w_ref[...], preferred_element_type=jnp.float32)
        + b_ref[...])


_encoder = pl.pallas_call(
    _enc_body,
    grid=(N // BN,),
    in_specs=[
        pl.BlockSpec((BN, 128), lambda i: (i, 0)),
        pl.BlockSpec((128, H), lambda i: (0, 0)),
        pl.BlockSpec((1, H), lambda i: (0, 0)),
    ],
    out_specs=pl.BlockSpec((BN, H), lambda i: (i, 0)),
    out_shape=jax.ShapeDtypeStruct((N, H), jnp.float32),
)


def _edge_body(a_ref, w_ref, b_ref, o_ref):
    o_ref[...] = (
        jnp.dot(a_ref[...], w_ref[...], preferred_element_type=jnp.float32)
        + b_ref[...])


_edge_embed = pl.pallas_call(
    _edge_body,
    grid=(EPAD // BE,),
    in_specs=[
        pl.BlockSpec((BE, D_EDGE), lambda i: (i, 0)),
        pl.BlockSpec((D_EDGE, H), lambda i: (0, 0)),
        pl.BlockSpec((1, H), lambda i: (0, 0)),
    ],
    out_specs=pl.BlockSpec((BE, H), lambda i: (i, 0)),
    out_shape=jax.ShapeDtypeStruct((EPAD, H), jnp.float32),
)


def _node_body(eps_ref, h_ref, a_ref, w1_ref, b1_ref, w2_ref, b2_ref, o_ref):
    z = h_ref[...] * eps_ref[0, 0] + a_ref[0] + a_ref[1]
    z = jnp.maximum(
        jnp.dot(z, w1_ref[...], preferred_element_type=jnp.float32)
        + b1_ref[...], 0.0)
    z = (jnp.dot(z, w2_ref[...], preferred_element_type=jnp.float32)
         + b2_ref[...])
    o_ref[...] = jnp.maximum(z, 0.0)


_node_update = pl.pallas_call(
    _node_body,
    grid=(N // BN,),
    in_specs=[
        pl.BlockSpec(memory_space=pltpu.SMEM),
        pl.BlockSpec((BN, H), lambda i: (i, 0)),
        pl.BlockSpec((NC, BN, H), lambda i: (0, i, 0)),
        pl.BlockSpec((H, H), lambda i: (0, 0)),
        pl.BlockSpec((1, H), lambda i: (0, 0)),
        pl.BlockSpec((H, H), lambda i: (0, 0)),
        pl.BlockSpec((1, H), lambda i: (0, 0)),
    ],
    out_specs=pl.BlockSpec((BN, H), lambda i: (i, 0)),
    out_shape=jax.ShapeDtypeStruct((N, H), jnp.float32),
)


def _head_body(h_ref, w0_ref, b0_ref, w1_ref, b1_ref, w2_ref, b2_ref, o_ref):
    o = jnp.maximum(
        jnp.dot(h_ref[...], w0_ref[...], preferred_element_type=jnp.float32)
        + b0_ref[...], 0.0)
    o = jnp.maximum(
        jnp.dot(o, w1_ref[...], preferred_element_type=jnp.float32)
        + b1_ref[...], 0.0)
    o_ref[...] = (
        jnp.dot(o, w2_ref[...], preferred_element_type=jnp.float32)
        + b2_ref[...])


_head = pl.pallas_call(
    _head_body,
    grid=(N // BN,),
    in_specs=[
        pl.BlockSpec((BN, H), lambda i: (i, 0)),
        pl.BlockSpec((H, H), lambda i: (0, 0)),
        pl.BlockSpec((1, H), lambda i: (0, 0)),
        pl.BlockSpec((H, H), lambda i: (0, 0)),
        pl.BlockSpec((1, H), lambda i: (0, 0)),
        pl.BlockSpec((H, H), lambda i: (0, 0)),
        pl.BlockSpec((1, H), lambda i: (0, 0)),
    ],
    out_specs=pl.BlockSpec((BN, H), lambda i: (i, 0)),
    out_shape=jax.ShapeDtypeStruct((N, H), jnp.float32),
)


def kernel(x, edge_index, edge_attr, y, params):
    p = params
    pad = EPAD - E
    src2d = jnp.concatenate(
        [edge_index[0], jnp.zeros((pad,), jnp.int32)]).reshape(EPAD // GROUP, GROUP)
    dst2d = jnp.concatenate(
        [edge_index[1], jnp.full((pad,), N, jnp.int32)]).reshape(EPAD // GROUP, GROUP)
    ea_pad = jnp.concatenate(
        [edge_attr, jnp.zeros((pad, D_EDGE), jnp.float32)], axis=0)
    zero_rows = jnp.zeros((RPW, H), jnp.float32)

    h = _encoder(x, p['enc_Wn'], p['enc_bn'].reshape(1, H))
    embs = []
    for l in range(L):
        wc = p['enc_We'] @ p[f'l{l}_elin_W']
        bc = p['enc_be'] @ p[f'l{l}_elin_W'] + p[f'l{l}_elin_b']
        embs.append(_edge_embed(ea_pad, wc, bc.reshape(1, H)))
    for l in range(L):
        agg2 = _sc_gather_scatter(h, embs[l], src2d, dst2d, zero_rows)
        g = p[f'l{l}_bn_g']
        w2 = p[f'l{l}_W2'] * g[None, :]
        b2 = p[f'l{l}_b2'] * g + p[f'l{l}_bn_b']
        epsm = (1.0 + p[f'l{l}_eps']).reshape(1, 1)
        h = _node_update(epsm, h, agg2, p[f'l{l}_W1'],
                         p[f'l{l}_b1'].reshape(1, H), w2, b2.reshape(1, H))

    w2p = jnp.pad(p['head_W2'], ((0, 0), (0, 127)))
    b2p = jnp.pad(p['head_b2'], (0, 127)).reshape(1, 128)
    o = _head(h, p['head_W0'], p['head_b0'].reshape(1, H),
              p['head_W1'], p['head_b1'].reshape(1, H), w2p, b2p)
    pred = o[:, :1]

    true_class = jnp.full((N,), -1, jnp.int32)
    true_label = jnp.where(y != -1.0, y, -1.0)
    return (pred, true_class, true_label)
